# baseline (speedup 1.0000x reference)
import functools

import jax
import jax.numpy as jnp
from jax import lax
from jax.experimental import pallas as pl
from jax.experimental.pallas import tpu as pltpu
from jax.experimental.pallas import tpu_sc as plsc

F32 = jnp.float32
N = 4096
E = 262144
D = 128
NH = 8
DH = 16

NC, NS = 2, 16
NW = NC * NS
EPT = E // NW
EB = 128
NB = EPT // EB
RPS = N // NS

def _mesh():
    return plsc.VectorSubcoreMesh(
        core_axis_name="c", subcore_axis_name="s", num_cores=NC, num_subcores=NS)


def _deg_body(dst2d, zeros_n, deg_out, dstv, ones, acc):
    c = lax.axis_index("c")
    s = lax.axis_index("s")
    wid = c * NS + s
    pltpu.sync_copy(dst2d.at[pl.ds(wid * NB, NB)], dstv)
    for i in range(EB // 16):
        ones[pl.ds(i * 16, 16)] = jnp.ones((16,), F32)
    pltpu.sync_copy(zeros_n.at[pl.ds(s * RPS, RPS)], acc.at[pl.ds(s * RPS, RPS)])
    plsc.subcore_barrier()

    def body(j, carry):
        pltpu.sync_copy(ones, acc.at[dstv.at[j]], add=True)
        return carry

    lax.fori_loop(0, NB, body, 0)
    plsc.subcore_barrier()
    pltpu.sync_copy(acc.at[pl.ds(s * RPS, RPS)], deg_out.at[c, pl.ds(s * RPS, RPS)])


@functools.cache
def _build_deg():
    return pl.kernel(
        _deg_body,
        out_type=jax.ShapeDtypeStruct((NC, N), F32),
        mesh=_mesh(),
        scratch_types=[
            pltpu.VMEM((NB, EB), jnp.int32),
            pltpu.VMEM((EB,), F32),
            pltpu.VMEM_SHARED((N,), F32),
        ],
    )


def _deg_call(dst2d, zeros_n):
    return _build_deg()(dst2d, zeros_n)


NBUF = 4
NR = NB // NBUF


def _scatter_body(src2d, dst2d, m_hbm, zeros_nd, s_out,
                  srcv, dstv, r0, r1, r2, r3, acc,
                  g0, g1, g2, g3, s0, s1, s2, s3):
    rows = (r0, r1, r2, r3)
    gsem = (g0, g1, g2, g3)
    ssem = (s0, s1, s2, s3)
    c = lax.axis_index("c")
    s = lax.axis_index("s")
    wid = c * NS + s
    pltpu.sync_copy(src2d.at[pl.ds(wid * NB, NB)], srcv)
    pltpu.sync_copy(dst2d.at[pl.ds(wid * NB, NB)], dstv)
    pltpu.sync_copy(zeros_nd.at[pl.ds(s * RPS, RPS)], acc.at[pl.ds(s * RPS, RPS)])
    plsc.subcore_barrier()

    def wait_gather(b):
        pltpu.make_async_copy(m_hbm.at[pl.ds(0, EB)], rows[b], gsem[b]).wait()

    def wait_scatter(b):
        pltpu.make_async_copy(rows[b], acc.at[pl.ds(0, EB)], ssem[b]).wait()

    for b in range(3):
        pltpu.async_copy(m_hbm.at[srcv.at[b]], rows[b], gsem[b])

    def body(r, carry):
        for b in range(NBUF):
            j = NBUF * r + b
            wait_gather(b)
            pltpu.async_copy(rows[b], acc.at[dstv.at[j]], ssem[b], add=True)
            bn = (b + 3) % NBUF
            if b == 0:
                @pl.when(r > 0)
                def _():
                    wait_scatter(bn)
                pltpu.async_copy(m_hbm.at[srcv.at[j + 3]], rows[bn], gsem[bn])
            else:
                @pl.when(r < NR - 1)
                def _():
                    wait_scatter(bn)
                    pltpu.async_copy(m_hbm.at[srcv.at[j + 3]], rows[bn], gsem[bn])
        return carry

    lax.fori_loop(0, NR, body, 0)
    for b in range(NBUF):
        wait_scatter(b)
    plsc.subcore_barrier()
    pltpu.sync_copy(acc.at[pl.ds(s * RPS, RPS)], s_out.at[c, pl.ds(s * RPS, RPS)])


@functools.cache
def _build_scatter():
    return pl.kernel(
        _scatter_body,
        out_type=jax.ShapeDtypeStruct((NC, N, D), F32),
        mesh=_mesh(),
        scratch_types=(
            [pltpu.VMEM((NB, EB), jnp.int32)] * 2
            + [pltpu.VMEM((EB, D), F32)] * NBUF
            + [pltpu.VMEM_SHARED((N, D), F32)]
            + [pltpu.SemaphoreType.DMA] * (2 * NBUF)
        ),
    )


def _scatter_call(src2d, dst2d, m_hbm, zeros_nd):
    return _build_scatter()(src2d, dst2d, m_hbm, zeros_nd)


_DN_T = (((1,), (1,)), ((), ()))


def _tc_u0_body(x_ref, w0_ref, u_ref):
    u_ref[...] = lax.dot_general(x_ref[...], w0_ref[...], _DN_T,
                                 preferred_element_type=F32)


def _tc_u0(x, w0):
    return pl.pallas_call(
        _tc_u0_body, out_shape=jax.ShapeDtypeStruct((N, D), F32))(x, w0)


def _tc0_body(parts_ref, u_ref, dinv_ref, m0_ref):
    degT = jnp.transpose(parts_ref[...])
    deg = jnp.sum(degT, axis=1, keepdims=True) + 1.0
    dinv = lax.rsqrt(deg)
    dinv_ref[...] = dinv
    m0_ref[...] = u_ref[...] * dinv


def _tc0(parts, u):
    return pl.pallas_call(
        _tc0_body,
        out_shape=(jax.ShapeDtypeStruct((N, 1), F32),
                   jax.ShapeDtypeStruct((N, D), F32)),
    )(parts, u)


def _tc_mid_body(sp_ref, mp_ref, dinv_ref, b_ref, gs_ref, be_ref, w_ref, out_ref):
    dinv = dinv_ref[...]
    z = (sp_ref[0] + sp_ref[1] + mp_ref[...]) * dinv + b_ref[...]
    h = jnp.maximum(z * gs_ref[...] + be_ref[...], 0.0)
    u = lax.dot_general(h, w_ref[...], _DN_T, preferred_element_type=F32)
    out_ref[...] = u * dinv


def _tc_mid(sp, mp, dinv, b, gs, be, w):
    return pl.pallas_call(
        _tc_mid_body,
        out_shape=jax.ShapeDtypeStruct((N, D), F32),
    )(sp, mp, dinv, b, gs, be, w)


def _tc_qkv_body(sp_ref, mp_ref, dinv_ref, b_ref, wqkv_ref, bqkv_ref,
                 q_ref, kt_ref, v_ref):
    h = (sp_ref[0] + sp_ref[1] + mp_ref[...]) * dinv_ref[...] + b_ref[...]
    qkv = lax.dot_general(h, wqkv_ref[...], _DN_T, preferred_element_type=F32)
    qkv = qkv + bqkv_ref[...]
    q_ref[...] = qkv[:, :D]
    kt_ref[...] = jnp.transpose(qkv[:, D:2 * D]).astype(jnp.bfloat16)
    v_ref[...] = qkv[:, 2 * D:].astype(jnp.bfloat16)


def _tc_qkv(sp, mp, dinv, b, wqkv, bqkv):
    return pl.pallas_call(
        _tc_qkv_body,
        out_shape=(jax.ShapeDtypeStruct((N, D), F32),
                   jax.ShapeDtypeStruct((D, N), jnp.bfloat16),
                   jax.ShapeDtypeStruct((N, D), jnp.bfloat16)),
    )(sp, mp, dinv, b, wqkv, bqkv)


QBLK = 1024
_LOG2E = 1.4426950408889634


def _tc_attn_body(q_ref, kt_ref, v_ref, wo_ref, bo_ref, wc1_ref, bc1_ref,
                  wc2_ref, bc2_ref, out_ref):
    q = q_ref[...]
    v = v_ref[...]
    ones_col = jnp.ones((N, 1), jnp.bfloat16)
    cols = []
    for h in range(NH):
        qh = (q[:, h * DH:(h + 1) * DH] * (0.25 * _LOG2E)).astype(jnp.bfloat16)
        kth = kt_ref[h * DH:(h + 1) * DH, :]
        sc = jnp.dot(qh, kth,
                     preferred_element_type=F32).astype(jnp.bfloat16)
        m = jnp.max(sc, axis=1, keepdims=True)
        ex = jnp.exp2(sc - m)
        vext = jnp.concatenate([v[:, h * DH:(h + 1) * DH], ones_col], axis=1)
        od = jnp.dot(ex, vext, preferred_element_type=F32)
        cols.append(od[:, :DH] / od[:, DH:])
    a = jnp.concatenate(cols, axis=1)
    a = lax.dot_general(a, wo_ref[...], _DN_T, preferred_element_type=F32) + bo_ref[...]
    z = lax.dot_general(a, wc1_ref[...], _DN_T, preferred_element_type=F32) + bc1_ref[...]
    z = jnp.maximum(z, 0.0)
    out_ref[...] = lax.dot_general(z, wc2_ref[...], _DN_T,
                                   preferred_element_type=F32) + bc2_ref[...]


def _tc_attn(q, kt, v, wo, bo, wc1, bc1, wc2, bc2):
    nq = N // QBLK
    whole = lambda shape: pl.BlockSpec(shape, lambda i: tuple(0 for _ in shape))
    return pl.pallas_call(
        _tc_attn_body,
        grid=(nq,),
        in_specs=[
            pl.BlockSpec((QBLK, D), lambda i: (i, 0)),
            whole((D, N)),
            whole((N, D)),
            whole((D, D)),
            whole((1, D)),
            whole((D // 2, D)),
            whole((1, D // 2)),
            whole((16, D // 2)),
            whole((1, 16)),
        ],
        out_specs=pl.BlockSpec((QBLK, 16), lambda i: (i, 0)),
        out_shape=jax.ShapeDtypeStruct((N, 16), F32),
    )(q, kt, v, wo, bo, wc1, bc1, wc2, bc2)


def kernel(x, edge_index, W0, b0, g0, be0, W1, b1, g1, be1, W2, b2,
           Wqkv, bqkv, Wo, bo, Wc1, bc1, Wc2, bc2):
    src2d = edge_index[0].reshape(E // EB, EB)
    dst2d = edge_index[1].reshape(E // EB, EB)
    zeros_n = jnp.zeros((N,), F32)
    zeros_nd = jnp.zeros((N, D), F32)

    bnc = 1.0 / jnp.sqrt(jnp.float32(1.0 + 1e-5))
    gs0 = (g0 * bnc).reshape(1, D)
    gs1 = (g1 * bnc).reshape(1, D)

    deg_parts = _deg_call(dst2d, zeros_n)
    u0 = _tc_u0(x, W0)
    dinv, m0 = _tc0(deg_parts, u0)

    s0 = _scatter_call(src2d, dst2d, m0, zeros_nd)
    m1 = _tc_mid(s0, m0, dinv, b0.reshape(1, D), gs0, be0.reshape(1, D), W1)

    s1 = _scatter_call(src2d, dst2d, m1, zeros_nd)
    m2 = _tc_mid(s1, m1, dinv, b1.reshape(1, D), gs1, be1.reshape(1, D), W2)

    s2 = _scatter_call(src2d, dst2d, m2, zeros_nd)
    q, kt, v = _tc_qkv(s2, m2, dinv, b2.reshape(1, D), Wqkv, bqkv.reshape(1, 3 * D))

    return _tc_attn(q, kt, v, Wo, bo.reshape(1, D), Wc1, bc1.reshape(1, D // 2),
                    Wc2, bc2.reshape(1, 16))


# speedup vs baseline: 1.1279x; 1.1279x over previous
import functools

import jax
import jax.numpy as jnp
from jax import lax
from jax.experimental import pallas as pl
from jax.experimental.pallas import tpu as pltpu
from jax.experimental.pallas import tpu_sc as plsc

F32 = jnp.float32
N = 4096
E = 262144
D = 128
NH = 8
DH = 16

NC, NS = 2, 16
NW = NC * NS
EPT = E // NW
EB = 128
NB = EPT // EB
RPS = N // NS

def _mesh():
    return plsc.VectorSubcoreMesh(
        core_axis_name="c", subcore_axis_name="s", num_cores=NC, num_subcores=NS)


def _deg_body(dst2d, zeros_n, deg_out, dstv, ones, acc, dsem):
    c = lax.axis_index("c")
    s = lax.axis_index("s")
    wid = c * NS + s
    pltpu.sync_copy(dst2d.at[pl.ds(wid * NB, NB)], dstv)
    for i in range(EB // 16):
        ones[pl.ds(i * 16, 16)] = jnp.ones((16,), F32)
    pltpu.sync_copy(zeros_n.at[pl.ds(s * RPS, RPS)], acc.at[pl.ds(s * RPS, RPS)])
    plsc.subcore_barrier()

    def body(r, carry):
        for b in range(8):
            pltpu.async_copy(ones, acc.at[dstv.at[8 * r + b]], dsem, add=True)
        for _ in range(8):
            pltpu.make_async_copy(ones, acc.at[pl.ds(0, EB)], dsem).wait()
        return carry

    lax.fori_loop(0, NB // 8, body, 0)
    plsc.subcore_barrier()
    pltpu.sync_copy(acc.at[pl.ds(s * RPS, RPS)], deg_out.at[c, pl.ds(s * RPS, RPS)])


@functools.cache
def _build_deg():
    return pl.kernel(
        _deg_body,
        out_type=jax.ShapeDtypeStruct((NC, N), F32),
        mesh=_mesh(),
        scratch_types=[
            pltpu.VMEM((NB, EB), jnp.int32),
            pltpu.VMEM((EB,), F32),
            pltpu.VMEM_SHARED((N,), F32),
            pltpu.SemaphoreType.DMA,
        ],
    )


def _deg_call(dst2d, zeros_n):
    return _build_deg()(dst2d, zeros_n)


NBUF = 4
NR = NB // NBUF


def _scatter_body(src2d, dst2d, m_hbm, zeros_nd, s_out,
                  srcv, dstv, r0, r1, r2, r3, acc,
                  g0, g1, g2, g3, s0, s1, s2, s3):
    rows = (r0, r1, r2, r3)
    gsem = (g0, g1, g2, g3)
    ssem = (s0, s1, s2, s3)
    c = lax.axis_index("c")
    s = lax.axis_index("s")
    wid = c * NS + s
    pltpu.sync_copy(src2d.at[pl.ds(wid * NB, NB)], srcv)
    pltpu.sync_copy(dst2d.at[pl.ds(wid * NB, NB)], dstv)
    pltpu.sync_copy(zeros_nd.at[pl.ds(s * RPS, RPS)], acc.at[pl.ds(s * RPS, RPS)])
    plsc.subcore_barrier()

    def wait_gather(b):
        pltpu.make_async_copy(m_hbm.at[pl.ds(0, EB)], rows[b], gsem[b]).wait()

    def wait_scatter(b):
        pltpu.make_async_copy(rows[b], acc.at[pl.ds(0, EB)], ssem[b]).wait()

    for b in range(3):
        pltpu.async_copy(m_hbm.at[srcv.at[b]], rows[b], gsem[b])

    def body(r, carry):
        for b in range(NBUF):
            j = NBUF * r + b
            wait_gather(b)
            pltpu.async_copy(rows[b], acc.at[dstv.at[j]], ssem[b], add=True)
            bn = (b + 3) % NBUF
            if b == 0:
                @pl.when(r > 0)
                def _():
                    wait_scatter(bn)
                pltpu.async_copy(m_hbm.at[srcv.at[j + 3]], rows[bn], gsem[bn])
            else:
                @pl.when(r < NR - 1)
                def _():
                    wait_scatter(bn)
                    pltpu.async_copy(m_hbm.at[srcv.at[j + 3]], rows[bn], gsem[bn])
        return carry

    lax.fori_loop(0, NR, body, 0)
    for b in range(NBUF):
        wait_scatter(b)
    plsc.subcore_barrier()
    pltpu.sync_copy(acc.at[pl.ds(s * RPS, RPS)], s_out.at[c, pl.ds(s * RPS, RPS)])


@functools.cache
def _build_scatter():
    return pl.kernel(
        _scatter_body,
        out_type=jax.ShapeDtypeStruct((NC, N, D), F32),
        mesh=_mesh(),
        scratch_types=(
            [pltpu.VMEM((NB, EB), jnp.int32)] * 2
            + [pltpu.VMEM((EB, D), F32)] * NBUF
            + [pltpu.VMEM_SHARED((N, D), F32)]
            + [pltpu.SemaphoreType.DMA] * (2 * NBUF)
        ),
    )


def _scatter_call(src2d, dst2d, m_hbm, zeros_nd):
    return _build_scatter()(src2d, dst2d, m_hbm, zeros_nd)


_DN_T = (((1,), (1,)), ((), ()))


def _tc_u0_body(x_ref, w0_ref, u_ref):
    u_ref[...] = lax.dot_general(x_ref[...], w0_ref[...], _DN_T,
                                 preferred_element_type=F32)


def _tc_u0(x, w0):
    return pl.pallas_call(
        _tc_u0_body, out_shape=jax.ShapeDtypeStruct((N, D), F32))(x, w0)


def _tc0_body(parts_ref, u_ref, dinv_ref, m0_ref):
    degT = jnp.transpose(parts_ref[...])
    deg = jnp.sum(degT, axis=1, keepdims=True) + 1.0
    dinv = lax.rsqrt(deg)
    dinv_ref[...] = dinv
    m0_ref[...] = u_ref[...] * dinv


def _tc0(parts, u):
    return pl.pallas_call(
        _tc0_body,
        out_shape=(jax.ShapeDtypeStruct((N, 1), F32),
                   jax.ShapeDtypeStruct((N, D), F32)),
    )(parts, u)


def _tc_mid_body(sp_ref, mp_ref, dinv_ref, b_ref, gs_ref, be_ref, w_ref, out_ref):
    dinv = dinv_ref[...]
    z = (sp_ref[0] + sp_ref[1] + mp_ref[...]) * dinv + b_ref[...]
    h = jnp.maximum(z * gs_ref[...] + be_ref[...], 0.0)
    u = lax.dot_general(h, w_ref[...], _DN_T, preferred_element_type=F32)
    out_ref[...] = u * dinv


def _tc_mid(sp, mp, dinv, b, gs, be, w):
    return pl.pallas_call(
        _tc_mid_body,
        out_shape=jax.ShapeDtypeStruct((N, D), F32),
    )(sp, mp, dinv, b, gs, be, w)


def _tc_qkv_body(sp_ref, mp_ref, dinv_ref, b_ref, wqkv_ref, bqkv_ref,
                 q_ref, kt_ref, v_ref):
    h = (sp_ref[0] + sp_ref[1] + mp_ref[...]) * dinv_ref[...] + b_ref[...]
    qkv = lax.dot_general(h, wqkv_ref[...], _DN_T, preferred_element_type=F32)
    qkv = qkv + bqkv_ref[...]
    q_ref[...] = qkv[:, :D]
    kt_ref[...] = jnp.transpose(qkv[:, D:2 * D]).astype(jnp.bfloat16)
    v_ref[...] = qkv[:, 2 * D:].astype(jnp.bfloat16)


def _tc_qkv(sp, mp, dinv, b, wqkv, bqkv):
    return pl.pallas_call(
        _tc_qkv_body,
        out_shape=(jax.ShapeDtypeStruct((N, D), F32),
                   jax.ShapeDtypeStruct((D, N), jnp.bfloat16),
                   jax.ShapeDtypeStruct((N, D), jnp.bfloat16)),
    )(sp, mp, dinv, b, wqkv, bqkv)


QBLK = 512
_LOG2E = 1.4426950408889634


def _tc_attn_body(q_ref, kt_ref, v_ref, wo_ref, bo_ref, wc1_ref, bc1_ref,
                  wc2_ref, bc2_ref, out_ref):
    q = q_ref[...]
    v = v_ref[...]
    ones_col = jnp.ones((N, 1), jnp.bfloat16)
    cols = []
    for h in range(NH):
        qh = (q[:, h * DH:(h + 1) * DH] * (0.25 * _LOG2E)).astype(jnp.bfloat16)
        kth = kt_ref[h * DH:(h + 1) * DH, :]
        sc = jnp.dot(qh, kth, preferred_element_type=F32)
        m = jnp.max(sc, axis=1, keepdims=True)
        ex = jnp.exp2((sc - m).astype(jnp.bfloat16))
        vext = jnp.concatenate([v[:, h * DH:(h + 1) * DH], ones_col], axis=1)
        od = jnp.dot(ex, vext, preferred_element_type=F32)
        cols.append(od[:, :DH] / od[:, DH:])
    a = jnp.concatenate(cols, axis=1)
    a = lax.dot_general(a, wo_ref[...], _DN_T, preferred_element_type=F32) + bo_ref[...]
    z = lax.dot_general(a, wc1_ref[...], _DN_T, preferred_element_type=F32) + bc1_ref[...]
    z = jnp.maximum(z, 0.0)
    out_ref[...] = lax.dot_general(z, wc2_ref[...], _DN_T,
                                   preferred_element_type=F32) + bc2_ref[...]


def _tc_attn(q, kt, v, wo, bo, wc1, bc1, wc2, bc2):
    nq = N // QBLK
    whole = lambda shape: pl.BlockSpec(shape, lambda i: tuple(0 for _ in shape))
    return pl.pallas_call(
        _tc_attn_body,
        grid=(nq,),
        in_specs=[
            pl.BlockSpec((QBLK, D), lambda i: (i, 0)),
            whole((D, N)),
            whole((N, D)),
            whole((D, D)),
            whole((1, D)),
            whole((D // 2, D)),
            whole((1, D // 2)),
            whole((16, D // 2)),
            whole((1, 16)),
        ],
        out_specs=pl.BlockSpec((QBLK, 16), lambda i: (i, 0)),
        out_shape=jax.ShapeDtypeStruct((N, 16), F32),
    )(q, kt, v, wo, bo, wc1, bc1, wc2, bc2)


def kernel(x, edge_index, W0, b0, g0, be0, W1, b1, g1, be1, W2, b2,
           Wqkv, bqkv, Wo, bo, Wc1, bc1, Wc2, bc2):
    src2d = edge_index[0].reshape(E // EB, EB)
    dst2d = edge_index[1].reshape(E // EB, EB)
    zeros_n = jnp.zeros((N,), F32)
    zeros_nd = jnp.zeros((N, D), F32)

    bnc = 1.0 / jnp.sqrt(jnp.float32(1.0 + 1e-5))
    gs0 = (g0 * bnc).reshape(1, D)
    gs1 = (g1 * bnc).reshape(1, D)

    deg_parts = _deg_call(dst2d, zeros_n)
    u0 = _tc_u0(x, W0)
    dinv, m0 = _tc0(deg_parts, u0)

    s0 = _scatter_call(src2d, dst2d, m0, zeros_nd)
    m1 = _tc_mid(s0, m0, dinv, b0.reshape(1, D), gs0, be0.reshape(1, D), W1)

    s1 = _scatter_call(src2d, dst2d, m1, zeros_nd)
    m2 = _tc_mid(s1, m1, dinv, b1.reshape(1, D), gs1, be1.reshape(1, D), W2)

    s2 = _scatter_call(src2d, dst2d, m2, zeros_nd)
    q, kt, v = _tc_qkv(s2, m2, dinv, b2.reshape(1, D), Wqkv, bqkv.reshape(1, 3 * D))

    return _tc_attn(q, kt, v, Wo, bo.reshape(1, D), Wc1, bc1.reshape(1, D // 2),
                    Wc2, bc2.reshape(1, 16))
